# Initial kernel scaffold; baseline (speedup 1.0000x reference)
import jax
import jax.numpy as jnp
from jax.experimental import pallas as pl


def kernel(x, edge_index, W1, b1, W2, b2, Wg, bg, Ws1, bs1, Ws2, bs2, Wo, bo):
    raise NotImplementedError("write your pallas kernel here")


# speedup vs baseline: 15.8469x; 15.8469x over previous
import functools

import jax
import jax.numpy as jnp
from jax import lax
from jax.experimental import pallas as pl
from jax.experimental.pallas import tpu as pltpu
from jax.experimental.pallas import tpu_sc as plsc

_NC = 2
_NS = 16
_CH = 80
_ZR = 200
_OCH = 1000
_DW = 16


def _feat_tc(x, W1, b1, W2, b2, Wg):
    n, d = x.shape
    h = W1.shape[1]
    r = 1000
    assert n % r == 0

    def body(x_ref, w1_ref, b1_ref, w2_ref, b2_ref, wg_ref, hf_ref, xw_ref):
        xb = x_ref[...]
        t = jnp.dot(xb, w1_ref[...], preferred_element_type=jnp.float32)
        t = jnp.maximum(t + b1_ref[...], 0.0)
        hf_ref[...] = jnp.dot(t, w2_ref[...],
                              preferred_element_type=jnp.float32) + b2_ref[...]
        xw_ref[...] = jnp.dot(xb, wg_ref[...],
                              preferred_element_type=jnp.float32)

    return pl.pallas_call(
        body,
        grid=(n // r,),
        in_specs=[
            pl.BlockSpec((r, d), lambda i: (i, 0)),
            pl.BlockSpec((d, h), lambda i: (0, 0)),
            pl.BlockSpec((1, h), lambda i: (0, 0)),
            pl.BlockSpec((h, h), lambda i: (0, 0)),
            pl.BlockSpec((1, h), lambda i: (0, 0)),
            pl.BlockSpec((d, h), lambda i: (0, 0)),
        ],
        out_specs=[pl.BlockSpec((r, h), lambda i: (i, 0)),
                   pl.BlockSpec((r, h), lambda i: (i, 0))],
        out_shape=[jax.ShapeDtypeStruct((n, h), jnp.float32),
                   jax.ShapeDtypeStruct((n, h), jnp.float32)],
    )(x, W1, b1.reshape(1, h), W2, b2.reshape(1, h), Wg)


def _sc_degree(dst, n, h):
    e = dst.shape[0]
    ept = e // (_NC * _NS)
    nchunk = ept // _CH
    nact = n // _OCH
    assert ept % _CH == 0 and n % _OCH == 0 and nact <= _NS
    mesh = plsc.VectorSubcoreMesh(core_axis_name="c", subcore_axis_name="s")

    @functools.partial(
        pl.kernel,
        out_type=jax.ShapeDtypeStruct((_NC, n, h), jnp.float32),
        mesh=mesh,
        scratch_types=[
            pltpu.VMEM((_CH,), jnp.int32),
            pltpu.VMEM((_CH, h), jnp.float32),
            pltpu.VMEM((_ZR, h), jnp.float32),
            pltpu.VMEM_SHARED((n, h), jnp.float32),
        ],
    )
    def k(dst_ref, out_ref, idx_v, ones_v, zb_v, acc_sh):
        c = lax.axis_index("c")
        s = lax.axis_index("s")
        wid = s * _NC + c

        @pl.loop(0, _ZR)
        def _(i):
            @pl.loop(0, h, step=16)
            def _(j):
                zb_v[pl.ds(i, 1), pl.ds(j, 16)] = jnp.zeros((1, 16), jnp.float32)

        @pl.loop(0, _CH)
        def _(i):
            @pl.loop(0, h, step=16)
            def _(j):
                ones_v[pl.ds(i, 1), pl.ds(j, 16)] = jnp.ones((1, 16), jnp.float32)

        @pl.when(s < nact)
        def _():
            @pl.loop(0, _OCH // _ZR)
            def _(k2):
                row0 = pl.multiple_of(s * _OCH + k2 * _ZR, 8)
                pltpu.sync_copy(zb_v, acc_sh.at[pl.ds(row0, _ZR)])

        plsc.subcore_barrier()

        @pl.loop(0, nchunk)
        def _(ci):
            base = pl.multiple_of(wid * ept + ci * _CH, _CH)
            pltpu.sync_copy(dst_ref.at[pl.ds(base, _CH)], idx_v)
            pltpu.sync_copy(ones_v, acc_sh.at[idx_v], add=True)

        plsc.subcore_barrier()

        @pl.when(s < nact)
        def _():
            row0 = pl.multiple_of(s * _OCH, 8)
            pltpu.sync_copy(acc_sh.at[pl.ds(row0, _OCH)],
                            out_ref.at[c, pl.ds(row0, _OCH)])

    return k(dst)


def _scale_tc(xw, d0, d1):
    n, h = xw.shape
    r = 1000
    assert n % r == 0

    def body(xw_ref, d0_ref, d1_ref, y_ref):
        deg = d0_ref[...] + d1_ref[...] + 1.0
        y_ref[...] = xw_ref[...] * lax.rsqrt(deg)

    blk = pl.BlockSpec((r, h), lambda i: (i, 0))
    return pl.pallas_call(
        body,
        grid=(n // r,),
        in_specs=[blk, blk, blk],
        out_specs=blk,
        out_shape=jax.ShapeDtypeStruct((n, h), jnp.float32),
    )(xw, d0, d1)


def _sc_scatter(src, dst, y):
    e = src.shape[0]
    n, h = y.shape
    ept = e // (_NC * _NS)
    nchunk = ept // _CH
    nact = n // _OCH
    assert ept % _CH == 0 and n % _OCH == 0 and nact <= _NS
    mesh = plsc.VectorSubcoreMesh(core_axis_name="c", subcore_axis_name="s")

    @functools.partial(
        pl.kernel,
        out_type=jax.ShapeDtypeStruct((_NC, n, h), jnp.float32),
        mesh=mesh,
        scratch_types=[
            pltpu.VMEM((_CH,), jnp.int32),
            pltpu.VMEM((_CH,), jnp.int32),
            pltpu.VMEM((_CH, h), jnp.float32),
            pltpu.VMEM((_ZR, h), jnp.float32),
            pltpu.VMEM_SHARED((n, h), jnp.float32),
            pltpu.SemaphoreType.DMA,
        ],
    )
    def k(src_ref, dst_ref, y_ref, out_ref, sidx, didx, rows, zb_v, acc_sh, sem):
        c = lax.axis_index("c")
        s = lax.axis_index("s")
        wid = s * _NC + c

        @pl.loop(0, _ZR)
        def _(i):
            @pl.loop(0, h, step=16)
            def _(j):
                zb_v[pl.ds(i, 1), pl.ds(j, 16)] = jnp.zeros((1, 16), jnp.float32)

        @pl.when(s < nact)
        def _():
            @pl.loop(0, _OCH // _ZR)
            def _(k2):
                row0 = pl.multiple_of(s * _OCH + k2 * _ZR, 8)
                pltpu.sync_copy(zb_v, acc_sh.at[pl.ds(row0, _ZR)])

        plsc.subcore_barrier()

        @pl.loop(0, nchunk)
        def _(ci):
            base = pl.multiple_of(wid * ept + ci * _CH, _CH)
            pltpu.sync_copy(src_ref.at[pl.ds(base, _CH)], sidx)
            pltpu.sync_copy(dst_ref.at[pl.ds(base, _CH)], didx)
            pltpu.async_copy(y_ref.at[sidx], rows, sem).wait()
            pltpu.sync_copy(rows, acc_sh.at[didx], add=True)

        plsc.subcore_barrier()

        @pl.when(s < nact)
        def _():
            row0 = pl.multiple_of(s * _OCH, 8)
            pltpu.sync_copy(acc_sh.at[pl.ds(row0, _OCH)],
                            out_ref.at[c, pl.ds(row0, _OCH)])

    return k(src, dst, y)


def _final_tc(u0, u1, y, d0, d1, hf, bg, Ws1, bs1, Ws2, bs2, WoT, WoB, bo):
    n, h = y.shape
    o = WoT.shape[1]
    r = 1000
    assert n % r == 0

    def body(u0_ref, u1_ref, y_ref, d0_ref, d1_ref, hf_ref, bg_ref,
             ws1_ref, bs1_ref, ws2_ref, bs2_ref, wot_ref, wob_ref, bo_ref,
             out_ref):
        deg = d0_ref[...] + d1_ref[...] + 1.0
        dinv = lax.rsqrt(deg)
        hagg = (u0_ref[...] + u1_ref[...] + y_ref[...]) * dinv + bg_ref[...]
        t = jnp.dot(hagg, ws1_ref[...], preferred_element_type=jnp.float32)
        t = jnp.maximum(t + bs1_ref[...], 0.0)
        hs = jnp.dot(t, ws2_ref[...],
                     preferred_element_type=jnp.float32) + bs2_ref[...]
        out_ref[...] = (
            jnp.dot(hf_ref[...], wot_ref[...], preferred_element_type=jnp.float32)
            + jnp.dot(hs, wob_ref[...], preferred_element_type=jnp.float32)
            + bo_ref[...])

    full = lambda a, b: pl.BlockSpec((a, b), lambda i: (0, 0))
    blk = lambda w: pl.BlockSpec((r, w), lambda i: (i, 0))
    return pl.pallas_call(
        body,
        grid=(n // r,),
        in_specs=[
            blk(h), blk(h), blk(h), blk(h), blk(h), blk(h),
            full(1, h), full(h, h), full(1, h), full(h, h), full(1, h),
            full(h, o), full(h, o), full(1, o),
        ],
        out_specs=pl.BlockSpec((r, o), lambda i: (i, 0)),
        out_shape=jax.ShapeDtypeStruct((n, o), jnp.float32),
    )(u0, u1, y, d0, d1, hf, bg.reshape(1, h), Ws1, bs1.reshape(1, h),
      Ws2, bs2.reshape(1, h), WoT, WoB, bo.reshape(1, o))


def kernel(x, edge_index, W1, b1, W2, b2, Wg, bg, Ws1, bs1, Ws2, bs2, Wo, bo):
    n = x.shape[0]
    h = W1.shape[1]
    src = edge_index[0]
    dst = edge_index[1]

    hf, xw = _feat_tc(x, W1, b1, W2, b2, Wg)
    degp = _sc_degree(dst, n, h)
    y = _scale_tc(xw, degp[0], degp[1])
    up = _sc_scatter(src, dst, y)
    return _final_tc(up[0], up[1], y, degp[0], degp[1], hf,
                     bg, Ws1, bs1, Ws2, bs2, Wo[:h], Wo[h:], bo)


# speedup vs baseline: 17.1585x; 1.0828x over previous
import functools

import jax
import jax.numpy as jnp
from jax import lax
from jax.experimental import pallas as pl
from jax.experimental.pallas import tpu as pltpu
from jax.experimental.pallas import tpu_sc as plsc

_NC = 2
_NS = 16
_CH = 80
_ZR = 200
_OCH = 1000
_DW = 16


def _feat_tc(x, W1, b1, W2, b2, Wg):
    n, d = x.shape
    h = W1.shape[1]
    r = 1000
    assert n % r == 0

    def body(x_ref, w1_ref, b1_ref, w2_ref, b2_ref, wg_ref, hf_ref, xw_ref):
        xb = x_ref[...]
        t = jnp.dot(xb, w1_ref[...], preferred_element_type=jnp.float32)
        t = jnp.maximum(t + b1_ref[...], 0.0)
        hf_ref[...] = jnp.dot(t, w2_ref[...],
                              preferred_element_type=jnp.float32) + b2_ref[...]
        xw_ref[...] = jnp.dot(xb, wg_ref[...],
                              preferred_element_type=jnp.float32)

    return pl.pallas_call(
        body,
        grid=(n // r,),
        in_specs=[
            pl.BlockSpec((r, d), lambda i: (i, 0)),
            pl.BlockSpec((d, h), lambda i: (0, 0)),
            pl.BlockSpec((1, h), lambda i: (0, 0)),
            pl.BlockSpec((h, h), lambda i: (0, 0)),
            pl.BlockSpec((1, h), lambda i: (0, 0)),
            pl.BlockSpec((d, h), lambda i: (0, 0)),
        ],
        out_specs=[pl.BlockSpec((r, h), lambda i: (i, 0)),
                   pl.BlockSpec((r, h), lambda i: (i, 0))],
        out_shape=[jax.ShapeDtypeStruct((n, h), jnp.float32),
                   jax.ShapeDtypeStruct((n, h), jnp.float32)],
    )(x, W1, b1.reshape(1, h), W2, b2.reshape(1, h), Wg)


def _sc_degree(dst, n):
    h = _DW
    e = dst.shape[0]
    ept = e // (_NC * _NS)
    nchunk = ept // _CH
    nact = n // _OCH
    assert ept % _CH == 0 and n % _OCH == 0 and nact <= _NS
    mesh = plsc.VectorSubcoreMesh(core_axis_name="c", subcore_axis_name="s")

    @functools.partial(
        pl.kernel,
        out_type=jax.ShapeDtypeStruct((_NC, n, h), jnp.float32),
        mesh=mesh,
        scratch_types=[
            pltpu.VMEM((_CH,), jnp.int32),
            pltpu.VMEM((_CH, h), jnp.float32),
            pltpu.VMEM((_ZR, h), jnp.float32),
            pltpu.VMEM_SHARED((n, h), jnp.float32),
        ],
    )
    def k(dst_ref, out_ref, idx_v, ones_v, zb_v, acc_sh):
        c = lax.axis_index("c")
        s = lax.axis_index("s")
        wid = s * _NC + c

        @pl.loop(0, _ZR)
        def _(i):
            @pl.loop(0, h, step=16)
            def _(j):
                zb_v[pl.ds(i, 1), pl.ds(j, 16)] = jnp.zeros((1, 16), jnp.float32)

        @pl.loop(0, _CH)
        def _(i):
            @pl.loop(0, h, step=16)
            def _(j):
                ones_v[pl.ds(i, 1), pl.ds(j, 16)] = jnp.ones((1, 16), jnp.float32)

        @pl.when(s < nact)
        def _():
            @pl.loop(0, _OCH // _ZR)
            def _(k2):
                row0 = pl.multiple_of(s * _OCH + k2 * _ZR, 8)
                pltpu.sync_copy(zb_v, acc_sh.at[pl.ds(row0, _ZR)])

        plsc.subcore_barrier()

        @pl.loop(0, nchunk)
        def _(ci):
            base = pl.multiple_of(wid * ept + ci * _CH, _CH)
            pltpu.sync_copy(dst_ref.at[pl.ds(base, _CH)], idx_v)
            pltpu.sync_copy(ones_v, acc_sh.at[idx_v], add=True)

        plsc.subcore_barrier()

        @pl.when(s < nact)
        def _():
            row0 = pl.multiple_of(s * _OCH, 8)
            pltpu.sync_copy(acc_sh.at[pl.ds(row0, _OCH)],
                            out_ref.at[c, pl.ds(row0, _OCH)])

    return k(dst)


def _scale_tc(xw, d0, d1):
    n, h = xw.shape
    r = 1000
    assert n % r == 0

    def body(xw_ref, d0_ref, d1_ref, y_ref):
        deg = d0_ref[...][:, :1] + d1_ref[...][:, :1] + 1.0
        y_ref[...] = xw_ref[...] * lax.rsqrt(deg)

    blk = pl.BlockSpec((r, h), lambda i: (i, 0))
    dblk = pl.BlockSpec((r, _DW), lambda i: (i, 0))
    return pl.pallas_call(
        body,
        grid=(n // r,),
        in_specs=[blk, dblk, dblk],
        out_specs=blk,
        out_shape=jax.ShapeDtypeStruct((n, h), jnp.float32),
    )(xw, d0, d1)


def _sc_scatter(src, dst, y):
    e = src.shape[0]
    n, h = y.shape
    ept = e // (_NC * _NS)
    nchunk = ept // _CH
    nact = n // _OCH
    assert ept % _CH == 0 and n % _OCH == 0 and nact <= _NS
    mesh = plsc.VectorSubcoreMesh(core_axis_name="c", subcore_axis_name="s")

    @functools.partial(
        pl.kernel,
        out_type=jax.ShapeDtypeStruct((_NC, n, h), jnp.float32),
        mesh=mesh,
        scratch_types=[
            pltpu.VMEM((_CH,), jnp.int32),
            pltpu.VMEM((_CH,), jnp.int32),
            pltpu.VMEM((_CH, h), jnp.float32),
            pltpu.VMEM((_ZR, h), jnp.float32),
            pltpu.VMEM_SHARED((n, h), jnp.float32),
            pltpu.SemaphoreType.DMA,
        ],
    )
    def k(src_ref, dst_ref, y_ref, out_ref, sidx, didx, rows, zb_v, acc_sh, sem):
        c = lax.axis_index("c")
        s = lax.axis_index("s")
        wid = s * _NC + c

        @pl.loop(0, _ZR)
        def _(i):
            @pl.loop(0, h, step=16)
            def _(j):
                zb_v[pl.ds(i, 1), pl.ds(j, 16)] = jnp.zeros((1, 16), jnp.float32)

        @pl.when(s < nact)
        def _():
            @pl.loop(0, _OCH // _ZR)
            def _(k2):
                row0 = pl.multiple_of(s * _OCH + k2 * _ZR, 8)
                pltpu.sync_copy(zb_v, acc_sh.at[pl.ds(row0, _ZR)])

        plsc.subcore_barrier()

        @pl.loop(0, nchunk)
        def _(ci):
            base = pl.multiple_of(wid * ept + ci * _CH, _CH)
            pltpu.sync_copy(src_ref.at[pl.ds(base, _CH)], sidx)
            pltpu.sync_copy(dst_ref.at[pl.ds(base, _CH)], didx)
            pltpu.async_copy(y_ref.at[sidx], rows, sem).wait()
            pltpu.sync_copy(rows, acc_sh.at[didx], add=True)

        plsc.subcore_barrier()

        @pl.when(s < nact)
        def _():
            row0 = pl.multiple_of(s * _OCH, 8)
            pltpu.sync_copy(acc_sh.at[pl.ds(row0, _OCH)],
                            out_ref.at[c, pl.ds(row0, _OCH)])

    return k(src, dst, y)


def _final_tc(u0, u1, y, d0, d1, hf, bg, Ws1, bs1, Ws2, bs2, WoT, WoB, bo):
    n, h = y.shape
    o = WoT.shape[1]
    r = 1000
    assert n % r == 0

    def body(u0_ref, u1_ref, y_ref, d0_ref, d1_ref, hf_ref, bg_ref,
             ws1_ref, bs1_ref, ws2_ref, bs2_ref, wot_ref, wob_ref, bo_ref,
             out_ref):
        deg = d0_ref[...][:, :1] + d1_ref[...][:, :1] + 1.0
        dinv = lax.rsqrt(deg)
        hagg = (u0_ref[...] + u1_ref[...] + y_ref[...]) * dinv + bg_ref[...]
        t = jnp.dot(hagg, ws1_ref[...], preferred_element_type=jnp.float32)
        t = jnp.maximum(t + bs1_ref[...], 0.0)
        hs = jnp.dot(t, ws2_ref[...],
                     preferred_element_type=jnp.float32) + bs2_ref[...]
        out_ref[...] = (
            jnp.dot(hf_ref[...], wot_ref[...], preferred_element_type=jnp.float32)
            + jnp.dot(hs, wob_ref[...], preferred_element_type=jnp.float32)
            + bo_ref[...])

    full = lambda a, b: pl.BlockSpec((a, b), lambda i: (0, 0))
    blk = lambda w: pl.BlockSpec((r, w), lambda i: (i, 0))
    return pl.pallas_call(
        body,
        grid=(n // r,),
        in_specs=[
            blk(h), blk(h), blk(h), blk(_DW), blk(_DW), blk(h),
            full(1, h), full(h, h), full(1, h), full(h, h), full(1, h),
            full(h, o), full(h, o), full(1, o),
        ],
        out_specs=pl.BlockSpec((r, o), lambda i: (i, 0)),
        out_shape=jax.ShapeDtypeStruct((n, o), jnp.float32),
    )(u0, u1, y, d0, d1, hf, bg.reshape(1, h), Ws1, bs1.reshape(1, h),
      Ws2, bs2.reshape(1, h), WoT, WoB, bo.reshape(1, o))


def kernel(x, edge_index, W1, b1, W2, b2, Wg, bg, Ws1, bs1, Ws2, bs2, Wo, bo):
    n = x.shape[0]
    h = W1.shape[1]
    src = edge_index[0]
    dst = edge_index[1]

    hf, xw = _feat_tc(x, W1, b1, W2, b2, Wg)
    degp = _sc_degree(dst, n)
    y = _scale_tc(xw, degp[0], degp[1])
    up = _sc_scatter(src, dst, y)
    return _final_tc(up[0], up[1], y, degp[0], degp[1], hf,
                     bg, Ws1, bs1, Ws2, bs2, Wo[:h], Wo[h:], bo)


# speedup vs baseline: 23.6535x; 1.3785x over previous
import functools

import jax
import jax.numpy as jnp
from jax import lax
from jax.experimental import pallas as pl
from jax.experimental.pallas import tpu as pltpu
from jax.experimental.pallas import tpu_sc as plsc

_NC = 2
_NS = 16
_CH = 80
_ZR = 200
_OCH = 1000
_DW = 16


def _feat_tc(x, W1, b1, W2, b2, Wg):
    n, d = x.shape
    h = W1.shape[1]
    r = 1000
    assert n % r == 0

    def body(x_ref, w1_ref, b1_ref, w2_ref, b2_ref, wg_ref, hf_ref, xw_ref):
        xb = x_ref[...]
        t = jnp.dot(xb, w1_ref[...], preferred_element_type=jnp.float32)
        t = jnp.maximum(t + b1_ref[...], 0.0)
        hf_ref[...] = jnp.dot(t, w2_ref[...],
                              preferred_element_type=jnp.float32) + b2_ref[...]
        xw_ref[...] = jnp.dot(xb, wg_ref[...],
                              preferred_element_type=jnp.float32)

    return pl.pallas_call(
        body,
        grid=(n // r,),
        in_specs=[
            pl.BlockSpec((r, d), lambda i: (i, 0)),
            pl.BlockSpec((d, h), lambda i: (0, 0)),
            pl.BlockSpec((1, h), lambda i: (0, 0)),
            pl.BlockSpec((h, h), lambda i: (0, 0)),
            pl.BlockSpec((1, h), lambda i: (0, 0)),
            pl.BlockSpec((d, h), lambda i: (0, 0)),
        ],
        out_specs=[pl.BlockSpec((r, h), lambda i: (i, 0)),
                   pl.BlockSpec((r, h), lambda i: (i, 0))],
        out_shape=[jax.ShapeDtypeStruct((n, h), jnp.float32),
                   jax.ShapeDtypeStruct((n, h), jnp.float32)],
    )(x, W1, b1.reshape(1, h), W2, b2.reshape(1, h), Wg)


def _sc_degree(dst, n):
    h = _DW
    e = dst.shape[0]
    ept = e // (_NC * _NS)
    nchunk = ept // _CH
    nact = n // _OCH
    assert ept % _CH == 0 and n % _OCH == 0 and nact <= _NS
    mesh = plsc.VectorSubcoreMesh(core_axis_name="c", subcore_axis_name="s")

    @functools.partial(
        pl.kernel,
        out_type=jax.ShapeDtypeStruct((_NC, n, h), jnp.float32),
        mesh=mesh,
        scratch_types=[
            pltpu.VMEM((_CH,), jnp.int32),
            pltpu.VMEM((_CH, h), jnp.float32),
            pltpu.VMEM((_ZR, h), jnp.float32),
            pltpu.VMEM_SHARED((n, h), jnp.float32),
        ],
    )
    def k(dst_ref, out_ref, idx_v, ones_v, zb_v, acc_sh):
        c = lax.axis_index("c")
        s = lax.axis_index("s")
        wid = s * _NC + c

        @pl.loop(0, _ZR)
        def _(i):
            @pl.loop(0, h, step=16)
            def _(j):
                zb_v[pl.ds(i, 1), pl.ds(j, 16)] = jnp.zeros((1, 16), jnp.float32)

        @pl.loop(0, _CH)
        def _(i):
            @pl.loop(0, h, step=16)
            def _(j):
                ones_v[pl.ds(i, 1), pl.ds(j, 16)] = jnp.ones((1, 16), jnp.float32)

        @pl.when(s < nact)
        def _():
            @pl.loop(0, _OCH // _ZR)
            def _(k2):
                row0 = pl.multiple_of(s * _OCH + k2 * _ZR, 8)
                pltpu.sync_copy(zb_v, acc_sh.at[pl.ds(row0, _ZR)])

        plsc.subcore_barrier()

        @pl.loop(0, nchunk)
        def _(ci):
            base = pl.multiple_of(wid * ept + ci * _CH, _CH)
            pltpu.sync_copy(dst_ref.at[pl.ds(base, _CH)], idx_v)
            pltpu.sync_copy(ones_v, acc_sh.at[idx_v], add=True)

        plsc.subcore_barrier()

        @pl.when(s < nact)
        def _():
            row0 = pl.multiple_of(s * _OCH, 8)
            pltpu.sync_copy(acc_sh.at[pl.ds(row0, _OCH)],
                            out_ref.at[c, pl.ds(row0, _OCH)])

    return k(dst)


def _scale_tc(xw, d0, d1):
    n, h = xw.shape
    r = 1000
    assert n % r == 0

    def body(xw_ref, d0_ref, d1_ref, y_ref):
        deg = d0_ref[...][:, :1] + d1_ref[...][:, :1] + 1.0
        y_ref[...] = xw_ref[...] * lax.rsqrt(deg)

    blk = pl.BlockSpec((r, h), lambda i: (i, 0))
    dblk = pl.BlockSpec((r, _DW), lambda i: (i, 0))
    return pl.pallas_call(
        body,
        grid=(n // r,),
        in_specs=[blk, dblk, dblk],
        out_specs=blk,
        out_shape=jax.ShapeDtypeStruct((n, h), jnp.float32),
    )(xw, d0, d1)


def _sc_scatter(src, dst, y):
    e = src.shape[0]
    n, h = y.shape
    ept = e // (_NC * _NS)
    nchunk = ept // _CH
    nact = n // _OCH
    assert ept % _CH == 0 and n % _OCH == 0 and nact <= _NS
    mesh = plsc.VectorSubcoreMesh(core_axis_name="c", subcore_axis_name="s")

    @functools.partial(
        pl.kernel,
        out_type=jax.ShapeDtypeStruct((_NC, n, h), jnp.float32),
        mesh=mesh,
        scratch_types=[
            pltpu.VMEM((_CH,), jnp.int32),
            pltpu.VMEM((_CH,), jnp.int32),
            pltpu.VMEM((_CH,), jnp.int32),
            pltpu.VMEM((_CH,), jnp.int32),
            pltpu.VMEM((_CH, h), jnp.float32),
            pltpu.VMEM((_CH, h), jnp.float32),
            pltpu.VMEM((_ZR, h), jnp.float32),
            pltpu.VMEM_SHARED((n, h), jnp.float32),
            pltpu.SemaphoreType.DMA,
            pltpu.SemaphoreType.DMA,
        ],
    )
    def k(src_ref, dst_ref, y_ref, out_ref, sidx0, sidx1, didx0, didx1,
          rows0, rows1, zb_v, acc_sh, sem0, sem1):
        c = lax.axis_index("c")
        s = lax.axis_index("s")
        wid = s * _NC + c
        sidx = (sidx0, sidx1)
        didx = (didx0, didx1)
        rows = (rows0, rows1)
        sem = (sem0, sem1)

        for b in range(2):
            base = pl.multiple_of(wid * ept + b * _CH, _CH)
            pltpu.sync_copy(src_ref.at[pl.ds(base, _CH)], sidx[b])
            pltpu.sync_copy(dst_ref.at[pl.ds(base, _CH)], didx[b])
            pltpu.async_copy(y_ref.at[sidx[b]], rows[b], sem[b])

        @pl.loop(0, _ZR)
        def _(i):
            @pl.loop(0, h, step=16)
            def _(j):
                zb_v[pl.ds(i, 1), pl.ds(j, 16)] = jnp.zeros((1, 16), jnp.float32)

        @pl.when(s < nact)
        def _():
            @pl.loop(0, _OCH // _ZR)
            def _(k2):
                row0 = pl.multiple_of(s * _OCH + k2 * _ZR, 8)
                pltpu.sync_copy(zb_v, acc_sh.at[pl.ds(row0, _ZR)])

        plsc.subcore_barrier()

        @pl.loop(0, nchunk, step=2)
        def _(ci):
            for b in range(2):
                cur = ci + b

                @pl.when(cur < nchunk)
                def _():
                    pltpu.make_async_copy(y_ref.at[sidx[b]], rows[b],
                                          sem[b]).wait()
                    pltpu.sync_copy(rows[b], acc_sh.at[didx[b]], add=True)

                    @pl.when(cur + 2 < nchunk)
                    def _():
                        base = pl.multiple_of(wid * ept + (cur + 2) * _CH, _CH)
                        pltpu.sync_copy(src_ref.at[pl.ds(base, _CH)], sidx[b])
                        pltpu.sync_copy(dst_ref.at[pl.ds(base, _CH)], didx[b])
                        pltpu.async_copy(y_ref.at[sidx[b]], rows[b], sem[b])

        plsc.subcore_barrier()

        @pl.when(s < nact)
        def _():
            row0 = pl.multiple_of(s * _OCH, 8)
            pltpu.sync_copy(acc_sh.at[pl.ds(row0, _OCH)],
                            out_ref.at[c, pl.ds(row0, _OCH)])

    return k(src, dst, y)


def _final_tc(u0, u1, y, d0, d1, hf, bg, Ws1, bs1, Ws2, bs2, WoT, WoB, bo):
    n, h = y.shape
    o = WoT.shape[1]
    r = 1000
    assert n % r == 0

    def body(u0_ref, u1_ref, y_ref, d0_ref, d1_ref, hf_ref, bg_ref,
             ws1_ref, bs1_ref, ws2_ref, bs2_ref, wot_ref, wob_ref, bo_ref,
             out_ref):
        deg = d0_ref[...][:, :1] + d1_ref[...][:, :1] + 1.0
        dinv = lax.rsqrt(deg)
        hagg = (u0_ref[...] + u1_ref[...] + y_ref[...]) * dinv + bg_ref[...]
        t = jnp.dot(hagg, ws1_ref[...], preferred_element_type=jnp.float32)
        t = jnp.maximum(t + bs1_ref[...], 0.0)
        hs = jnp.dot(t, ws2_ref[...],
                     preferred_element_type=jnp.float32) + bs2_ref[...]
        out_ref[...] = (
            jnp.dot(hf_ref[...], wot_ref[...], preferred_element_type=jnp.float32)
            + jnp.dot(hs, wob_ref[...], preferred_element_type=jnp.float32)
            + bo_ref[...])

    full = lambda a, b: pl.BlockSpec((a, b), lambda i: (0, 0))
    blk = lambda w: pl.BlockSpec((r, w), lambda i: (i, 0))
    return pl.pallas_call(
        body,
        grid=(n // r,),
        in_specs=[
            blk(h), blk(h), blk(h), blk(_DW), blk(_DW), blk(h),
            full(1, h), full(h, h), full(1, h), full(h, h), full(1, h),
            full(h, o), full(h, o), full(1, o),
        ],
        out_specs=pl.BlockSpec((r, o), lambda i: (i, 0)),
        out_shape=jax.ShapeDtypeStruct((n, o), jnp.float32),
    )(u0, u1, y, d0, d1, hf, bg.reshape(1, h), Ws1, bs1.reshape(1, h),
      Ws2, bs2.reshape(1, h), WoT, WoB, bo.reshape(1, o))


def kernel(x, edge_index, W1, b1, W2, b2, Wg, bg, Ws1, bs1, Ws2, bs2, Wo, bo):
    n = x.shape[0]
    h = W1.shape[1]
    src = edge_index[0]
    dst = edge_index[1]

    hf, xw = _feat_tc(x, W1, b1, W2, b2, Wg)
    degp = _sc_degree(dst, n)
    y = _scale_tc(xw, degp[0], degp[1])
    up = _sc_scatter(src, dst, y)
    return _final_tc(up[0], up[1], y, degp[0], degp[1], hf,
                     bg, Ws1, bs1, Ws2, bs2, Wo[:h], Wo[h:], bo)


# speedup vs baseline: 26.2505x; 1.1098x over previous
import functools

import jax
import jax.numpy as jnp
from jax import lax
from jax.experimental import pallas as pl
from jax.experimental.pallas import tpu as pltpu
from jax.experimental.pallas import tpu_sc as plsc

_NC = 2
_NS = 16
_CH = 80
_ZR = 200
_OCH = 1000
_DW = 16


def _feat_tc(x, W1, b1, W2, b2, Wg):
    n, d = x.shape
    h = W1.shape[1]
    r = 1000
    assert n % r == 0

    def body(x_ref, w1_ref, b1_ref, w2_ref, b2_ref, wg_ref, hf_ref, xw_ref):
        xb = x_ref[...]
        t = jnp.dot(xb, w1_ref[...], preferred_element_type=jnp.float32)
        t = jnp.maximum(t + b1_ref[...], 0.0)
        hf_ref[...] = jnp.dot(t, w2_ref[...],
                              preferred_element_type=jnp.float32) + b2_ref[...]
        xw_ref[...] = jnp.dot(xb, wg_ref[...],
                              preferred_element_type=jnp.float32)

    return pl.pallas_call(
        body,
        grid=(n // r,),
        in_specs=[
            pl.BlockSpec((r, d), lambda i: (i, 0)),
            pl.BlockSpec((d, h), lambda i: (0, 0)),
            pl.BlockSpec((1, h), lambda i: (0, 0)),
            pl.BlockSpec((h, h), lambda i: (0, 0)),
            pl.BlockSpec((1, h), lambda i: (0, 0)),
            pl.BlockSpec((d, h), lambda i: (0, 0)),
        ],
        out_specs=[pl.BlockSpec((r, h), lambda i: (i, 0)),
                   pl.BlockSpec((r, h), lambda i: (i, 0))],
        out_shape=[jax.ShapeDtypeStruct((n, h), jnp.float32),
                   jax.ShapeDtypeStruct((n, h), jnp.float32)],
    )(x, W1, b1.reshape(1, h), W2, b2.reshape(1, h), Wg)


def _sc_degree(dst, n):
    h = _DW
    e = dst.shape[0]
    ept = e // (_NC * _NS)
    nchunk = ept // _CH
    nact = n // _OCH
    assert ept % _CH == 0 and n % _OCH == 0 and nact <= _NS
    mesh = plsc.VectorSubcoreMesh(core_axis_name="c", subcore_axis_name="s")

    @functools.partial(
        pl.kernel,
        out_type=jax.ShapeDtypeStruct((_NC, n, h), jnp.float32),
        mesh=mesh,
        scratch_types=[
            pltpu.VMEM((_CH,), jnp.int32),
            pltpu.VMEM((_CH,), jnp.int32),
            pltpu.VMEM((_CH, h), jnp.float32),
            pltpu.VMEM((_ZR, h), jnp.float32),
            pltpu.VMEM_SHARED((n, h), jnp.float32),
            pltpu.SemaphoreType.DMA,
            pltpu.SemaphoreType.DMA,
        ],
    )
    def k(dst_ref, out_ref, idx0, idx1, ones_v, zb_v, acc_sh, sem0, sem1):
        c = lax.axis_index("c")
        s = lax.axis_index("s")
        wid = s * _NC + c
        idx = (idx0, idx1)
        sem = (sem0, sem1)

        for b in range(2):
            base = pl.multiple_of(wid * ept + b * _CH, _CH)
            pltpu.async_copy(dst_ref.at[pl.ds(base, _CH)], idx[b], sem[b])

        @pl.loop(0, _ZR)
        def _(i):
            @pl.loop(0, h, step=16)
            def _(j):
                zb_v[pl.ds(i, 1), pl.ds(j, 16)] = jnp.zeros((1, 16), jnp.float32)

        @pl.loop(0, _CH)
        def _(i):
            @pl.loop(0, h, step=16)
            def _(j):
                ones_v[pl.ds(i, 1), pl.ds(j, 16)] = jnp.ones((1, 16), jnp.float32)

        @pl.when(s < nact)
        def _():
            @pl.loop(0, _OCH // _ZR)
            def _(k2):
                row0 = pl.multiple_of(s * _OCH + k2 * _ZR, 8)
                pltpu.sync_copy(zb_v, acc_sh.at[pl.ds(row0, _ZR)])

        plsc.subcore_barrier()

        @pl.loop(0, nchunk, step=2)
        def _(ci):
            for b in range(2):
                cur = ci + b

                @pl.when(cur < nchunk)
                def _():
                    pltpu.make_async_copy(
                        dst_ref.at[pl.ds(pl.multiple_of(0, _CH), _CH)],
                        idx[b], sem[b]).wait()
                    pltpu.sync_copy(ones_v, acc_sh.at[idx[b]], add=True)

                    @pl.when(cur + 2 < nchunk)
                    def _():
                        base = pl.multiple_of(wid * ept + (cur + 2) * _CH, _CH)
                        pltpu.async_copy(dst_ref.at[pl.ds(base, _CH)],
                                         idx[b], sem[b])

        plsc.subcore_barrier()

        @pl.when(s < nact)
        def _():
            row0 = pl.multiple_of(s * _OCH, 8)
            pltpu.sync_copy(acc_sh.at[pl.ds(row0, _OCH)],
                            out_ref.at[c, pl.ds(row0, _OCH)])

    return k(dst)


def _scale_tc(xw, d0, d1):
    n, h = xw.shape
    r = 1000
    assert n % r == 0

    def body(xw_ref, d0_ref, d1_ref, y_ref):
        deg = d0_ref[...][:, :1] + d1_ref[...][:, :1] + 1.0
        y_ref[...] = xw_ref[...] * lax.rsqrt(deg)

    blk = pl.BlockSpec((r, h), lambda i: (i, 0))
    dblk = pl.BlockSpec((r, _DW), lambda i: (i, 0))
    return pl.pallas_call(
        body,
        grid=(n // r,),
        in_specs=[blk, dblk, dblk],
        out_specs=blk,
        out_shape=jax.ShapeDtypeStruct((n, h), jnp.float32),
    )(xw, d0, d1)


def _sc_scatter(src, dst, y):
    e = src.shape[0]
    n, h = y.shape
    ept = e // (_NC * _NS)
    nchunk = ept // _CH
    nact = n // _OCH
    assert ept % _CH == 0 and n % _OCH == 0 and nact <= _NS
    mesh = plsc.VectorSubcoreMesh(core_axis_name="c", subcore_axis_name="s")

    @functools.partial(
        pl.kernel,
        out_type=jax.ShapeDtypeStruct((_NC, n, h), jnp.float32),
        mesh=mesh,
        scratch_types=[
            pltpu.VMEM((_CH,), jnp.int32),
            pltpu.VMEM((_CH,), jnp.int32),
            pltpu.VMEM((_CH,), jnp.int32),
            pltpu.VMEM((_CH,), jnp.int32),
            pltpu.VMEM((_CH, h), jnp.float32),
            pltpu.VMEM((_CH, h), jnp.float32),
            pltpu.VMEM((_ZR, h), jnp.float32),
            pltpu.VMEM_SHARED((n, h), jnp.float32),
            pltpu.SemaphoreType.DMA,
            pltpu.SemaphoreType.DMA,
        ],
    )
    def k(src_ref, dst_ref, y_ref, out_ref, sidx0, sidx1, didx0, didx1,
          rows0, rows1, zb_v, acc_sh, sem0, sem1):
        c = lax.axis_index("c")
        s = lax.axis_index("s")
        wid = s * _NC + c
        sidx = (sidx0, sidx1)
        didx = (didx0, didx1)
        rows = (rows0, rows1)
        sem = (sem0, sem1)

        for b in range(2):
            base = pl.multiple_of(wid * ept + b * _CH, _CH)
            pltpu.sync_copy(src_ref.at[pl.ds(base, _CH)], sidx[b])
            pltpu.sync_copy(dst_ref.at[pl.ds(base, _CH)], didx[b])
            pltpu.async_copy(y_ref.at[sidx[b]], rows[b], sem[b])

        @pl.loop(0, _ZR)
        def _(i):
            @pl.loop(0, h, step=16)
            def _(j):
                zb_v[pl.ds(i, 1), pl.ds(j, 16)] = jnp.zeros((1, 16), jnp.float32)

        @pl.when(s < nact)
        def _():
            @pl.loop(0, _OCH // _ZR)
            def _(k2):
                row0 = pl.multiple_of(s * _OCH + k2 * _ZR, 8)
                pltpu.sync_copy(zb_v, acc_sh.at[pl.ds(row0, _ZR)])

        plsc.subcore_barrier()

        @pl.loop(0, nchunk, step=2)
        def _(ci):
            for b in range(2):
                cur = ci + b

                @pl.when(cur < nchunk)
                def _():
                    pltpu.make_async_copy(y_ref.at[sidx[b]], rows[b],
                                          sem[b]).wait()
                    pltpu.sync_copy(rows[b], acc_sh.at[didx[b]], add=True)

                    @pl.when(cur + 2 < nchunk)
                    def _():
                        base = pl.multiple_of(wid * ept + (cur + 2) * _CH, _CH)
                        pltpu.sync_copy(src_ref.at[pl.ds(base, _CH)], sidx[b])
                        pltpu.sync_copy(dst_ref.at[pl.ds(base, _CH)], didx[b])
                        pltpu.async_copy(y_ref.at[sidx[b]], rows[b], sem[b])

        plsc.subcore_barrier()

        @pl.when(s < nact)
        def _():
            row0 = pl.multiple_of(s * _OCH, 8)
            pltpu.sync_copy(acc_sh.at[pl.ds(row0, _OCH)],
                            out_ref.at[c, pl.ds(row0, _OCH)])

    return k(src, dst, y)


def _final_tc(u0, u1, y, d0, d1, hf, bg, Ws1, bs1, Ws2, bs2, WoT, WoB, bo):
    n, h = y.shape
    o = WoT.shape[1]
    r = 1000
    assert n % r == 0

    def body(u0_ref, u1_ref, y_ref, d0_ref, d1_ref, hf_ref, bg_ref,
             ws1_ref, bs1_ref, ws2_ref, bs2_ref, wot_ref, wob_ref, bo_ref,
             out_ref):
        deg = d0_ref[...][:, :1] + d1_ref[...][:, :1] + 1.0
        dinv = lax.rsqrt(deg)
        hagg = (u0_ref[...] + u1_ref[...] + y_ref[...]) * dinv + bg_ref[...]
        t = jnp.dot(hagg, ws1_ref[...], preferred_element_type=jnp.float32)
        t = jnp.maximum(t + bs1_ref[...], 0.0)
        hs = jnp.dot(t, ws2_ref[...],
                     preferred_element_type=jnp.float32) + bs2_ref[...]
        out_ref[...] = (
            jnp.dot(hf_ref[...], wot_ref[...], preferred_element_type=jnp.float32)
            + jnp.dot(hs, wob_ref[...], preferred_element_type=jnp.float32)
            + bo_ref[...])

    full = lambda a, b: pl.BlockSpec((a, b), lambda i: (0, 0))
    blk = lambda w: pl.BlockSpec((r, w), lambda i: (i, 0))
    return pl.pallas_call(
        body,
        grid=(n // r,),
        in_specs=[
            blk(h), blk(h), blk(h), blk(_DW), blk(_DW), blk(h),
            full(1, h), full(h, h), full(1, h), full(h, h), full(1, h),
            full(h, o), full(h, o), full(1, o),
        ],
        out_specs=pl.BlockSpec((r, o), lambda i: (i, 0)),
        out_shape=jax.ShapeDtypeStruct((n, o), jnp.float32),
    )(u0, u1, y, d0, d1, hf, bg.reshape(1, h), Ws1, bs1.reshape(1, h),
      Ws2, bs2.reshape(1, h), WoT, WoB, bo.reshape(1, o))


def kernel(x, edge_index, W1, b1, W2, b2, Wg, bg, Ws1, bs1, Ws2, bs2, Wo, bo):
    n = x.shape[0]
    h = W1.shape[1]
    src = edge_index[0]
    dst = edge_index[1]

    hf, xw = _feat_tc(x, W1, b1, W2, b2, Wg)
    degp = _sc_degree(dst, n)
    y = _scale_tc(xw, degp[0], degp[1])
    up = _sc_scatter(src, dst, y)
    return _final_tc(up[0], up[1], y, degp[0], degp[1], hf,
                     bg, Ws1, bs1, Ws2, bs2, Wo[:h], Wo[h:], bo)
